# baseline (speedup 1.0000x reference)
import functools

import jax
import jax.numpy as jnp
from jax import lax
from jax.experimental import pallas as pl
from jax.experimental.pallas import tpu as pltpu
from jax.experimental.pallas import tpu_sc as plsc

NU = 10000
NI = 10000
E = 160000
E2 = E // 2
D = 256
H = 4
DH = 64
DEA = 16
HID = 128

NCORE = 2
NSUB = 16
NW = NCORE * NSUB

C = 1600
G = C // 16

TBASE = (0, 1, 0, 1)

NB = 1000
NBLK = NU // NB


_sc_mesh = plsc.VectorSubcoreMesh(core_axis_name="c", subcore_axis_name="s")
_sc_params = pltpu.CompilerParams(
    needs_layout_passes=False, use_tc_tiling_on_sc=False
)


def _sc_body(xtab, eatab, srcp, dstc, z2, z1, aggx, agge, degp,
             acc2, acc1, sidx, didx, rows, sem):
    wid = lax.axis_index("s") * NCORE + lax.axis_index("c")
    iota = lax.iota(jnp.int32, 16)

    def scatter_chunk(c, _):
        def grp(g, _):
            d = didx[pl.ds(g * 16, 16)]
            e0 = g * 16 + iota
            for f in range(8):
                cf = jnp.full((16,), f, jnp.int32)
                vals = plsc.load_gather(rows, [e0, cf])
                plsc.addupdate_scatter(acc2, [d, cf], vals)
            return _

        return lax.fori_loop(0, G, grp, _, unroll=4)

    for r in range(4):
        pltpu.sync_copy(z2, acc2)
        sb = TBASE[r] * (32 * 10000) + wid * 10000

        def chunk_body(c, _, r=r, sb=sb):
            off = r * E + c * C
            pltpu.sync_copy(srcp.at[pl.ds(off, C)], sidx)
            cp = pltpu.async_copy(xtab.at[pl.ds(sb, 10000)].at[sidx], rows, sem)
            pltpu.sync_copy(dstc.at[pl.ds(off, C)], didx)
            cp.wait()
            return scatter_chunk(c, _)

        lax.fori_loop(0, E // C, chunk_body, 0)
        pltpu.sync_copy(acc2, aggx.at[pl.ds((r * 32 + wid) * 10000, 10000), :])

    @pl.when(wid < 16)
    def _():
        r = wid // 4
        h = (wid // 2) % 2
        half = wid % 2
        row0 = (r * 2 + h) * E + half * E2
        doff0 = r * E + half * E2
        pltpu.sync_copy(z2, acc2)

        def chunk_body2(c, _):
            pltpu.sync_copy(eatab.at[pl.ds(row0 + c * C, C), :], rows)
            pltpu.sync_copy(dstc.at[pl.ds(doff0 + c * C, C)], didx)
            return scatter_chunk(c, _)

        lax.fori_loop(0, E2 // C, chunk_body2, 0)
        pltpu.sync_copy(acc2, agge.at[pl.ds(wid * 10000, 10000), :])

    @pl.when((wid >= 16) & (wid < 24))
    def _():
        u = wid - 16
        r = u // 2
        half = u % 2
        doff0 = r * E + half * E2
        pltpu.sync_copy(z1, acc1)
        ones = jnp.ones((16,), jnp.float32)

        def chunk_body3(c, _):
            pltpu.sync_copy(dstc.at[pl.ds(doff0 + c * C, C)], didx)

            def grp3(g, _):
                d = didx[pl.ds(g * 16, 16)]
                plsc.addupdate_scatter(acc1, [d], ones)
                return _

            return lax.fori_loop(0, G, grp3, _, unroll=8)

        lax.fori_loop(0, E2 // C, chunk_body3, 0)
        pltpu.sync_copy(acc1, degp.at[pl.ds(u * 10000, 10000)])


_sc_aggregate = functools.partial(
    pl.kernel,
    mesh=_sc_mesh,
    compiler_params=_sc_params,
    out_type=[
        jax.ShapeDtypeStruct((4 * 32 * 10000, 8), jnp.float32),
        jax.ShapeDtypeStruct((16 * 10000, 8), jnp.float32),
        jax.ShapeDtypeStruct((8 * 10000,), jnp.float32),
    ],
    scratch_types=[
        pltpu.VMEM((10000, 8), jnp.float32),
        pltpu.VMEM((10000,), jnp.float32),
        pltpu.VMEM((C,), jnp.int32),
        pltpu.VMEM((C,), jnp.int32),
        pltpu.VMEM((C, 8), jnp.float32),
        pltpu.SemaphoreType.DMA,
    ],
)(_sc_body)


def _p1_body(a0, e0, d0, a1, e1, d1, W0, We0, W1r, We1r, W1, b1, w2r,
             z0o, z1o, s0o, s1o, acc):
    i = pl.program_id(0)

    @pl.when(i == 0)
    def _():
        acc[0] = 0.0
        acc[1] = 0.0

    rd0 = 1.0 / jnp.maximum(d0[...], 1.0)
    rd1 = 1.0 / jnp.maximum(d1[...], 1.0)
    z0 = (jnp.dot(a0[...], W0[...], preferred_element_type=jnp.float32)
          + jnp.dot(e0[...], We0[...], preferred_element_type=jnp.float32)) * rd0
    z1 = (jnp.dot(a1[...], W1r[...], preferred_element_type=jnp.float32)
          + jnp.dot(e1[...], We1r[...], preferred_element_type=jnp.float32)) * rd1
    h0 = jnp.tanh(jnp.dot(z0, W1[...], preferred_element_type=jnp.float32) + b1[...])
    h1 = jnp.tanh(jnp.dot(z1, W1[...], preferred_element_type=jnp.float32) + b1[...])
    acc[0] += jnp.sum(h0 * w2r[...])
    acc[1] += jnp.sum(h1 * w2r[...])
    z0o[...] = z0
    z1o[...] = z1

    @pl.when(i == NBLK - 1)
    def _():
        s0o[...] = jnp.full((1, 128), acc[0], jnp.float32)
        s1o[...] = jnp.full((1, 128), acc[1], jnp.float32)


def _p2_body(z0, z1, s0, s1, o):
    t0 = s0[0, 0] * (1.0 / NU)
    t1 = s1[0, 0] * (1.0 / NU)
    m = jnp.maximum(t0, t1)
    e0 = jnp.exp(t0 - m)
    e1 = jnp.exp(t1 - m)
    b0 = e0 / (e0 + e1)
    b1 = e1 / (e0 + e1)
    o[...] = b0 * z0[...] + b1 * z1[...]


def _dense_pair(a0, e0, d0, a1, e1, d1, W0, We0, W1r, We1r, W1, b1, w2):
    row = lambda i: (i, 0)
    const = lambda i: (0, 0)
    z0, z1, s0, s1 = pl.pallas_call(
        _p1_body,
        grid=(NBLK,),
        in_specs=[
            pl.BlockSpec((NB, D), row),
            pl.BlockSpec((NB, DEA), row),
            pl.BlockSpec((NB, 1), row),
            pl.BlockSpec((NB, D), row),
            pl.BlockSpec((NB, DEA), row),
            pl.BlockSpec((NB, 1), row),
            pl.BlockSpec((D, D), const),
            pl.BlockSpec((DEA, D), const),
            pl.BlockSpec((D, D), const),
            pl.BlockSpec((DEA, D), const),
            pl.BlockSpec((D, HID), const),
            pl.BlockSpec((1, HID), const),
            pl.BlockSpec((1, HID), const),
        ],
        out_specs=[
            pl.BlockSpec((NB, D), row),
            pl.BlockSpec((NB, D), row),
            pl.BlockSpec((1, 128), const),
            pl.BlockSpec((1, 128), const),
        ],
        out_shape=[
            jax.ShapeDtypeStruct((NU, D), jnp.float32),
            jax.ShapeDtypeStruct((NU, D), jnp.float32),
            jax.ShapeDtypeStruct((1, 128), jnp.float32),
            jax.ShapeDtypeStruct((1, 128), jnp.float32),
        ],
        scratch_shapes=[pltpu.SMEM((2,), jnp.float32)],
    )(a0, e0, d0.reshape(NU, 1), a1, e1, d1.reshape(NU, 1),
      W0, We0, W1r, We1r, W1, b1.reshape(1, HID), w2.reshape(1, HID))

    out = pl.pallas_call(
        _p2_body,
        grid=(NBLK,),
        in_specs=[
            pl.BlockSpec((NB, D), row),
            pl.BlockSpec((NB, D), row),
            pl.BlockSpec((1, 128), const),
            pl.BlockSpec((1, 128), const),
        ],
        out_specs=pl.BlockSpec((NB, D), row),
        out_shape=jax.ShapeDtypeStruct((NU, D), jnp.float32),
    )(z0, z1, s0, s1)
    return out.reshape(NU, H, DH)


def kernel(x_user, x_item, ei_follows, ei_boughtby, ei_buys, ei_similar,
           ea_follows, ea_boughtby, ea_buys, ea_similar,
           W_follows, We_follows, W_boughtby, We_boughtby,
           W_buys, We_buys, W_similar, We_similar,
           W1_u, b1_u, w2_u, W1_i, b1_i, w2_i):
    xu3 = x_user.reshape(NU, 32, 8).transpose(1, 0, 2).reshape(-1, 8)
    xi3 = x_item.reshape(NI, 32, 8).transpose(1, 0, 2).reshape(-1, 8)
    xtab = jnp.concatenate([xu3, xi3], axis=0)

    eas = [ea_follows, ea_boughtby, ea_buys, ea_similar]
    eatab = jnp.concatenate(
        [ea.reshape(E, 2, 8).transpose(1, 0, 2).reshape(-1, 8) for ea in eas],
        axis=0)

    eis = [ei_follows, ei_boughtby, ei_buys, ei_similar]
    srcp = jnp.concatenate([ei[0] for ei in eis], axis=0)
    dstc = jnp.concatenate([ei[1] for ei in eis], axis=0)

    z2 = jnp.zeros((10000, 8), jnp.float32)
    z1 = jnp.zeros((10000,), jnp.float32)

    aggx, agge, degp = _sc_aggregate(xtab, eatab, srcp, dstc, z2, z1)

    aggx4 = aggx.reshape(4, 32, NU, 8).transpose(0, 2, 1, 3).reshape(4, NU, D)
    agge4 = (agge.reshape(4, 2, 2, NU, 8).sum(axis=2)
             .transpose(0, 2, 1, 3).reshape(4, NU, DEA))
    deg4 = degp.reshape(4, 2, NU).sum(axis=1)

    out_user = _dense_pair(aggx4[0], agge4[0], deg4[0],
                           aggx4[1], agge4[1], deg4[1],
                           W_follows, We_follows, W_boughtby, We_boughtby,
                           W1_u, b1_u, w2_u)
    out_item = _dense_pair(aggx4[2], agge4[2], deg4[2],
                           aggx4[3], agge4[3], deg4[3],
                           W_buys, We_buys, W_similar, We_similar,
                           W1_i, b1_i, w2_i)
    return (out_user, out_item)


# speedup vs baseline: 1.8286x; 1.8286x over previous
import functools

import jax
import jax.numpy as jnp
from jax import lax
from jax.experimental import pallas as pl
from jax.experimental.pallas import tpu as pltpu
from jax.experimental.pallas import tpu_sc as plsc

NU = 10000
NI = 10000
E = 160000
E2 = E // 2
D = 256
H = 4
DH = 64
DEA = 16
HID = 128

NCORE = 2
NSUB = 16
NW = NCORE * NSUB

C = 1600
G = C // 16

TBASE = (0, 1, 0, 1)

NB = 1000
NBLK = NU // NB


_sc_mesh = plsc.VectorSubcoreMesh(core_axis_name="c", subcore_axis_name="s")
_sc_params = pltpu.CompilerParams(
    needs_layout_passes=False, use_tc_tiling_on_sc=False
)


def _sc_body(xtab, eatab, srcp, dstc, z2, z1, aggx, agge, degp,
             acc2, acc1, sidx, didx, rows, onesb, sem):
    wid = lax.axis_index("s") * NCORE + lax.axis_index("c")
    sid = lax.axis_index("s")
    a2b = sid * 10000
    a1b = sid * 10000

    for r in range(4):
        pltpu.sync_copy(z2, acc2.at[pl.ds(a2b, 10000), :])
        sb = TBASE[r] * (32 * 10000) + wid * 10000

        def chunk_body(c, _, r=r, sb=sb):
            off = r * E + c * C
            pltpu.sync_copy(srcp.at[pl.ds(off, C)], sidx)
            cp = pltpu.async_copy(xtab.at[pl.ds(sb, 10000)].at[sidx], rows, sem)
            pltpu.sync_copy(dstc.at[pl.ds(off, C)], didx)
            cp.wait()
            pltpu.sync_copy(rows, acc2.at[pl.ds(a2b, 10000)].at[didx], add=True)
            return _

        lax.fori_loop(0, E // C, chunk_body, 0)
        pltpu.sync_copy(acc2.at[pl.ds(a2b, 10000), :],
                        aggx.at[pl.ds((r * 32 + wid) * 10000, 10000), :])

    @pl.when(wid < 16)
    def _():
        r = wid // 4
        h = (wid // 2) % 2
        half = wid % 2
        row0 = (r * 2 + h) * E + half * E2
        doff0 = r * E + half * E2
        pltpu.sync_copy(z2, acc2.at[pl.ds(a2b, 10000), :])

        def chunk_body2(c, _):
            pltpu.sync_copy(eatab.at[pl.ds(row0 + c * C, C), :], rows)
            pltpu.sync_copy(dstc.at[pl.ds(doff0 + c * C, C)], didx)
            pltpu.sync_copy(rows, acc2.at[pl.ds(a2b, 10000)].at[didx], add=True)
            return _

        lax.fori_loop(0, E2 // C, chunk_body2, 0)
        pltpu.sync_copy(acc2.at[pl.ds(a2b, 10000), :],
                        agge.at[pl.ds(wid * 10000, 10000), :])

    @pl.when((wid >= 16) & (wid < 24))
    def _():
        u = wid - 16
        r = u // 2
        half = u % 2
        doff0 = r * E + half * E2
        pltpu.sync_copy(z1, acc1.at[pl.ds(a1b, 10000)])

        def fill(g, _):
            onesb[pl.ds(g * 16, 16)] = jnp.ones((16,), jnp.float32)
            return _

        lax.fori_loop(0, G, fill, 0, unroll=8)

        def chunk_body3(c, _):
            pltpu.sync_copy(dstc.at[pl.ds(doff0 + c * C, C)], didx)
            pltpu.sync_copy(onesb, acc1.at[pl.ds(a1b, 10000)].at[didx], add=True)
            return _

        lax.fori_loop(0, E2 // C, chunk_body3, 0)
        pltpu.sync_copy(acc1.at[pl.ds(a1b, 10000)],
                        degp.at[pl.ds(u * 10000, 10000)])


_sc_aggregate = functools.partial(
    pl.kernel,
    mesh=_sc_mesh,
    compiler_params=_sc_params,
    out_type=[
        jax.ShapeDtypeStruct((4 * 32 * 10000, 8), jnp.float32),
        jax.ShapeDtypeStruct((16 * 10000, 8), jnp.float32),
        jax.ShapeDtypeStruct((8 * 10000,), jnp.float32),
    ],
    scratch_types=[
        pltpu.VMEM_SHARED((NSUB * 10000, 8), jnp.float32),
        pltpu.VMEM_SHARED((NSUB * 10000,), jnp.float32),
        pltpu.VMEM((C,), jnp.int32),
        pltpu.VMEM((C,), jnp.int32),
        pltpu.VMEM((C, 8), jnp.float32),
        pltpu.VMEM((C,), jnp.float32),
        pltpu.SemaphoreType.DMA,
    ],
)(_sc_body)


def _p1_body(a0, e0, d0, a1, e1, d1, W0, We0, W1r, We1r, W1, b1, w2r,
             z0o, z1o, s0o, s1o, acc):
    i = pl.program_id(0)

    @pl.when(i == 0)
    def _():
        acc[0] = 0.0
        acc[1] = 0.0

    rd0 = 1.0 / jnp.maximum(d0[...], 1.0)
    rd1 = 1.0 / jnp.maximum(d1[...], 1.0)
    z0 = (jnp.dot(a0[...], W0[...], preferred_element_type=jnp.float32)
          + jnp.dot(e0[...], We0[...], preferred_element_type=jnp.float32)) * rd0
    z1 = (jnp.dot(a1[...], W1r[...], preferred_element_type=jnp.float32)
          + jnp.dot(e1[...], We1r[...], preferred_element_type=jnp.float32)) * rd1
    h0 = jnp.tanh(jnp.dot(z0, W1[...], preferred_element_type=jnp.float32) + b1[...])
    h1 = jnp.tanh(jnp.dot(z1, W1[...], preferred_element_type=jnp.float32) + b1[...])
    acc[0] += jnp.sum(h0 * w2r[...])
    acc[1] += jnp.sum(h1 * w2r[...])
    z0o[...] = z0
    z1o[...] = z1

    @pl.when(i == NBLK - 1)
    def _():
        s0o[...] = jnp.full((1, 128), acc[0], jnp.float32)
        s1o[...] = jnp.full((1, 128), acc[1], jnp.float32)


def _p2_body(z0, z1, s0, s1, o):
    t0 = s0[0, 0] * (1.0 / NU)
    t1 = s1[0, 0] * (1.0 / NU)
    m = jnp.maximum(t0, t1)
    e0 = jnp.exp(t0 - m)
    e1 = jnp.exp(t1 - m)
    b0 = e0 / (e0 + e1)
    b1 = e1 / (e0 + e1)
    o[...] = b0 * z0[...] + b1 * z1[...]


def _dense_pair(a0, e0, d0, a1, e1, d1, W0, We0, W1r, We1r, W1, b1, w2):
    row = lambda i: (i, 0)
    const = lambda i: (0, 0)
    z0, z1, s0, s1 = pl.pallas_call(
        _p1_body,
        grid=(NBLK,),
        in_specs=[
            pl.BlockSpec((NB, D), row),
            pl.BlockSpec((NB, DEA), row),
            pl.BlockSpec((NB, 1), row),
            pl.BlockSpec((NB, D), row),
            pl.BlockSpec((NB, DEA), row),
            pl.BlockSpec((NB, 1), row),
            pl.BlockSpec((D, D), const),
            pl.BlockSpec((DEA, D), const),
            pl.BlockSpec((D, D), const),
            pl.BlockSpec((DEA, D), const),
            pl.BlockSpec((D, HID), const),
            pl.BlockSpec((1, HID), const),
            pl.BlockSpec((1, HID), const),
        ],
        out_specs=[
            pl.BlockSpec((NB, D), row),
            pl.BlockSpec((NB, D), row),
            pl.BlockSpec((1, 128), const),
            pl.BlockSpec((1, 128), const),
        ],
        out_shape=[
            jax.ShapeDtypeStruct((NU, D), jnp.float32),
            jax.ShapeDtypeStruct((NU, D), jnp.float32),
            jax.ShapeDtypeStruct((1, 128), jnp.float32),
            jax.ShapeDtypeStruct((1, 128), jnp.float32),
        ],
        scratch_shapes=[pltpu.SMEM((2,), jnp.float32)],
    )(a0, e0, d0.reshape(NU, 1), a1, e1, d1.reshape(NU, 1),
      W0, We0, W1r, We1r, W1, b1.reshape(1, HID), w2.reshape(1, HID))

    out = pl.pallas_call(
        _p2_body,
        grid=(NBLK,),
        in_specs=[
            pl.BlockSpec((NB, D), row),
            pl.BlockSpec((NB, D), row),
            pl.BlockSpec((1, 128), const),
            pl.BlockSpec((1, 128), const),
        ],
        out_specs=pl.BlockSpec((NB, D), row),
        out_shape=jax.ShapeDtypeStruct((NU, D), jnp.float32),
    )(z0, z1, s0, s1)
    return out.reshape(NU, H, DH)


def kernel(x_user, x_item, ei_follows, ei_boughtby, ei_buys, ei_similar,
           ea_follows, ea_boughtby, ea_buys, ea_similar,
           W_follows, We_follows, W_boughtby, We_boughtby,
           W_buys, We_buys, W_similar, We_similar,
           W1_u, b1_u, w2_u, W1_i, b1_i, w2_i):
    xu3 = x_user.reshape(NU, 32, 8).transpose(1, 0, 2).reshape(-1, 8)
    xi3 = x_item.reshape(NI, 32, 8).transpose(1, 0, 2).reshape(-1, 8)
    xtab = jnp.concatenate([xu3, xi3], axis=0)

    eas = [ea_follows, ea_boughtby, ea_buys, ea_similar]
    eatab = jnp.concatenate(
        [ea.reshape(E, 2, 8).transpose(1, 0, 2).reshape(-1, 8) for ea in eas],
        axis=0)

    eis = [ei_follows, ei_boughtby, ei_buys, ei_similar]
    srcp = jnp.concatenate([ei[0] for ei in eis], axis=0)
    dstc = jnp.concatenate([ei[1] for ei in eis], axis=0)

    z2 = jnp.zeros((10000, 8), jnp.float32)
    z1 = jnp.zeros((10000,), jnp.float32)

    aggx, agge, degp = _sc_aggregate(xtab, eatab, srcp, dstc, z2, z1)

    aggx4 = aggx.reshape(4, 32, NU, 8).transpose(0, 2, 1, 3).reshape(4, NU, D)
    agge4 = (agge.reshape(4, 2, 2, NU, 8).sum(axis=2)
             .transpose(0, 2, 1, 3).reshape(4, NU, DEA))
    deg4 = degp.reshape(4, 2, NU).sum(axis=1)

    out_user = _dense_pair(aggx4[0], agge4[0], deg4[0],
                           aggx4[1], agge4[1], deg4[1],
                           W_follows, We_follows, W_boughtby, We_boughtby,
                           W1_u, b1_u, w2_u)
    out_item = _dense_pair(aggx4[2], agge4[2], deg4[2],
                           aggx4[3], agge4[3], deg4[3],
                           W_buys, We_buys, W_similar, We_similar,
                           W1_i, b1_i, w2_i)
    return (out_user, out_item)


# speedup vs baseline: 2.3230x; 1.2704x over previous
import functools

import jax
import jax.numpy as jnp
from jax import lax
from jax.experimental import pallas as pl
from jax.experimental.pallas import tpu as pltpu
from jax.experimental.pallas import tpu_sc as plsc

NU = 10000
NI = 10000
E = 160000
E2 = E // 2
D = 256
H = 4
DH = 64
DEA = 16
HID = 128

NCORE = 2
NSUB = 16
NW = NCORE * NSUB

C = 1600
G = C // 16

TBASE = (0, 1, 0, 1)

NB = 1000
NBLK = NU // NB


_sc_mesh = plsc.VectorSubcoreMesh(core_axis_name="c", subcore_axis_name="s")
_sc_params = pltpu.CompilerParams(
    needs_layout_passes=False, use_tc_tiling_on_sc=False
)


def _sc_body(xtab,
             src_f, dst_f, src_b, dst_b, src_u, dst_u, src_s, dst_s,
             ea_f, ea_b, ea_u, ea_s, z2, z1, aggx, aggeP, degp,
             acc2, acc1, sidx, didx, rows, onesb, sem):
    wid = lax.axis_index("s") * NCORE + lax.axis_index("c")
    sid = lax.axis_index("s")
    a2b = sid * 10000
    a1b = sid * 10000
    w8 = wid * 8

    srcs = (src_f, src_b, src_u, src_s)
    dsts = (dst_f, dst_b, dst_u, dst_s)
    eas = (ea_f, ea_b, ea_u, ea_s)

    for r in range(4):
        sb = TBASE[r] * (32 * 10000) + wid * 10000
        src_r = srcs[r]
        dst_r = dsts[r]
        pltpu.sync_copy(z2, acc2.at[pl.ds(a2b, 10000), :])

        def chunk_body(c, _, sb=sb, src_r=src_r, dst_r=dst_r):
            off = c * C
            pltpu.sync_copy(src_r.at[pl.ds(off, C)], sidx)
            cp = pltpu.async_copy(
                xtab.at[pl.ds(sb, 10000)].at[sidx], rows, sem)
            pltpu.sync_copy(dst_r.at[pl.ds(off, C)], didx)
            cp.wait()
            pltpu.sync_copy(rows, acc2.at[pl.ds(a2b, 10000)].at[didx], add=True)
            return _

        lax.fori_loop(0, E // C, chunk_body, 0)
        pltpu.sync_copy(acc2.at[pl.ds(a2b, 10000), :],
                        aggx.at[pl.ds(r * 10000, 10000), pl.ds(w8, 8)])

    for r in range(4):
        @pl.when(wid // 4 == r)
        def _(r=r):
            h = (wid // 2) % 2
            half = wid % 2
            eoff0 = half * E2
            ea_r = eas[r]
            dst_r = dsts[r]
            pltpu.sync_copy(z2, acc2.at[pl.ds(a2b, 10000), :])

            def chunk_body2(c, _):
                pltpu.sync_copy(
                    ea_r.at[pl.ds(eoff0 + c * C, C), pl.ds(h * 8, 8)], rows)
                pltpu.sync_copy(dst_r.at[pl.ds(eoff0 + c * C, C)], didx)
                pltpu.sync_copy(rows, acc2.at[pl.ds(a2b, 10000)].at[didx],
                                add=True)
                return _

            lax.fori_loop(0, E2 // C, chunk_body2, 0)
            pltpu.sync_copy(
                acc2.at[pl.ds(a2b, 10000), :],
                aggeP.at[pl.ds(half * 40000 + r * 10000, 10000),
                         pl.ds(h * 8, 8)])

    for r in range(4):
        @pl.when((wid >= 16) & (wid < 24) & ((wid - 16) // 2 == r))
        def _(r=r):
            half = wid % 2
            doff0 = half * E2
            dst_r = dsts[r]
            pltpu.sync_copy(z1, acc1.at[pl.ds(a1b, 10000)])

            def fill(g, _):
                onesb[pl.ds(g * 16, 16)] = jnp.ones((16,), jnp.float32)
                return _

            lax.fori_loop(0, G, fill, 0, unroll=8)

            def chunk_body3(c, _):
                pltpu.sync_copy(dst_r.at[pl.ds(doff0 + c * C, C)], didx)
                pltpu.sync_copy(onesb, acc1.at[pl.ds(a1b, 10000)].at[didx],
                                add=True)
                return _

            lax.fori_loop(0, E2 // C, chunk_body3, 0)
            pltpu.sync_copy(acc1.at[pl.ds(a1b, 10000)],
                            degp.at[pl.ds((r * 2 + half) * 10000, 10000)])


_sc_aggregate = functools.partial(
    pl.kernel,
    mesh=_sc_mesh,
    compiler_params=_sc_params,
    out_type=[
        jax.ShapeDtypeStruct((4 * 10000, 256), jnp.float32),
        jax.ShapeDtypeStruct((2 * 4 * 10000, 16), jnp.float32),
        jax.ShapeDtypeStruct((8 * 10000,), jnp.float32),
    ],
    scratch_types=[
        pltpu.VMEM_SHARED((NSUB * 10000, 8), jnp.float32),
        pltpu.VMEM_SHARED((NSUB * 10000,), jnp.float32),
        pltpu.VMEM((C,), jnp.int32),
        pltpu.VMEM((C,), jnp.int32),
        pltpu.VMEM((C, 8), jnp.float32),
        pltpu.VMEM((C,), jnp.float32),
        pltpu.SemaphoreType.DMA,
    ],
)(_sc_body)


def _p1_body(a0, e0, d0, a1, e1, d1, W0, We0, W1r, We1r, W1, b1, w2r,
             z0o, z1o, s0o, s1o, acc):
    i = pl.program_id(0)

    @pl.when(i == 0)
    def _():
        acc[0] = 0.0
        acc[1] = 0.0

    rd0 = 1.0 / jnp.maximum(d0[...], 1.0)
    rd1 = 1.0 / jnp.maximum(d1[...], 1.0)
    z0 = (jnp.dot(a0[...], W0[...], preferred_element_type=jnp.float32)
          + jnp.dot(e0[...], We0[...], preferred_element_type=jnp.float32)) * rd0
    z1 = (jnp.dot(a1[...], W1r[...], preferred_element_type=jnp.float32)
          + jnp.dot(e1[...], We1r[...], preferred_element_type=jnp.float32)) * rd1
    h0 = jnp.tanh(jnp.dot(z0, W1[...], preferred_element_type=jnp.float32) + b1[...])
    h1 = jnp.tanh(jnp.dot(z1, W1[...], preferred_element_type=jnp.float32) + b1[...])
    acc[0] += jnp.sum(h0 * w2r[...])
    acc[1] += jnp.sum(h1 * w2r[...])
    z0o[...] = z0
    z1o[...] = z1

    @pl.when(i == NBLK - 1)
    def _():
        s0o[...] = jnp.full((1, 128), acc[0], jnp.float32)
        s1o[...] = jnp.full((1, 128), acc[1], jnp.float32)


def _p2_body(z0, z1, s0, s1, o):
    t0 = s0[0, 0] * (1.0 / NU)
    t1 = s1[0, 0] * (1.0 / NU)
    m = jnp.maximum(t0, t1)
    e0 = jnp.exp(t0 - m)
    e1 = jnp.exp(t1 - m)
    b0 = e0 / (e0 + e1)
    b1 = e1 / (e0 + e1)
    o[...] = b0 * z0[...] + b1 * z1[...]


def _dense_pair(a0, e0, d0, a1, e1, d1, W0, We0, W1r, We1r, W1, b1, w2):
    row = lambda i: (i, 0)
    const = lambda i: (0, 0)
    z0, z1, s0, s1 = pl.pallas_call(
        _p1_body,
        grid=(NBLK,),
        in_specs=[
            pl.BlockSpec((NB, D), row),
            pl.BlockSpec((NB, DEA), row),
            pl.BlockSpec((NB, 1), row),
            pl.BlockSpec((NB, D), row),
            pl.BlockSpec((NB, DEA), row),
            pl.BlockSpec((NB, 1), row),
            pl.BlockSpec((D, D), const),
            pl.BlockSpec((DEA, D), const),
            pl.BlockSpec((D, D), const),
            pl.BlockSpec((DEA, D), const),
            pl.BlockSpec((D, HID), const),
            pl.BlockSpec((1, HID), const),
            pl.BlockSpec((1, HID), const),
        ],
        out_specs=[
            pl.BlockSpec((NB, D), row),
            pl.BlockSpec((NB, D), row),
            pl.BlockSpec((1, 128), const),
            pl.BlockSpec((1, 128), const),
        ],
        out_shape=[
            jax.ShapeDtypeStruct((NU, D), jnp.float32),
            jax.ShapeDtypeStruct((NU, D), jnp.float32),
            jax.ShapeDtypeStruct((1, 128), jnp.float32),
            jax.ShapeDtypeStruct((1, 128), jnp.float32),
        ],
        scratch_shapes=[pltpu.SMEM((2,), jnp.float32)],
    )(a0, e0, d0.reshape(NU, 1), a1, e1, d1.reshape(NU, 1),
      W0, We0, W1r, We1r, W1, b1.reshape(1, HID), w2.reshape(1, HID))

    out = pl.pallas_call(
        _p2_body,
        grid=(NBLK,),
        in_specs=[
            pl.BlockSpec((NB, D), row),
            pl.BlockSpec((NB, D), row),
            pl.BlockSpec((1, 128), const),
            pl.BlockSpec((1, 128), const),
        ],
        out_specs=pl.BlockSpec((NB, D), row),
        out_shape=jax.ShapeDtypeStruct((NU, D), jnp.float32),
    )(z0, z1, s0, s1)
    return out.reshape(NU, H, DH)


def kernel(x_user, x_item, ei_follows, ei_boughtby, ei_buys, ei_similar,
           ea_follows, ea_boughtby, ea_buys, ea_similar,
           W_follows, We_follows, W_boughtby, We_boughtby,
           W_buys, We_buys, W_similar, We_similar,
           W1_u, b1_u, w2_u, W1_i, b1_i, w2_i):
    xu3 = x_user.reshape(NU, 32, 8).transpose(1, 0, 2).reshape(-1, 8)
    xi3 = x_item.reshape(NI, 32, 8).transpose(1, 0, 2).reshape(-1, 8)
    xtab = jnp.concatenate([xu3, xi3], axis=0)

    z2 = jnp.zeros((10000, 8), jnp.float32)
    z1 = jnp.zeros((10000,), jnp.float32)

    aggx, aggeP, degp = _sc_aggregate(
        xtab,
        ei_follows[0], ei_follows[1], ei_boughtby[0], ei_boughtby[1],
        ei_buys[0], ei_buys[1], ei_similar[0], ei_similar[1],
        ea_follows, ea_boughtby, ea_buys, ea_similar, z2, z1)

    aggx4 = aggx.reshape(4, NU, D)
    agge4 = aggeP.reshape(2, 4, NU, DEA).sum(axis=0)
    deg4 = degp.reshape(4, 2, NU).sum(axis=1)

    out_user = _dense_pair(aggx4[0], agge4[0], deg4[0],
                           aggx4[1], agge4[1], deg4[1],
                           W_follows, We_follows, W_boughtby, We_boughtby,
                           W1_u, b1_u, w2_u)
    out_item = _dense_pair(aggx4[2], agge4[2], deg4[2],
                           aggx4[3], agge4[3], deg4[3],
                           W_buys, We_buys, W_similar, We_similar,
                           W1_i, b1_i, w2_i)
    return (out_user, out_item)


# speedup vs baseline: 3.3321x; 1.4344x over previous
import functools

import jax
import jax.numpy as jnp
from jax import lax
from jax.experimental import pallas as pl
from jax.experimental.pallas import tpu as pltpu
from jax.experimental.pallas import tpu_sc as plsc

NU = 10000
NI = 10000
E = 160000
E2 = E // 2
D = 256
H = 4
DH = 64
DEA = 16
HID = 128

NCORE = 2
NSUB = 16
NW = NCORE * NSUB

C = 2000
G = C // 16

TBASE = (0, 1, 0, 1)

NB = 1000
NBLK = NU // NB


_sc_mesh = plsc.VectorSubcoreMesh(core_axis_name="c", subcore_axis_name="s")
_sc_params = pltpu.CompilerParams(
    needs_layout_passes=False, use_tc_tiling_on_sc=False
)


def _sc_body(xtab,
             src_f, dst_f, src_b, dst_b, src_u, dst_u, src_s, dst_s,
             ea_f, ea_b, ea_u, ea_s, z2, z1, aggx, aggeP, degp,
             acc2, acc1, sidx0, sidx1, didx0, didx1, rows0, rows1, onesb,
             sem0, sem1):
    wid = lax.axis_index("s") * NCORE + lax.axis_index("c")
    sid = lax.axis_index("s")
    a2b = sid * 10000
    a1b = (sid - 8) * 10000
    w8 = wid * 8

    srcs = (src_f, src_b, src_u, src_s)
    dsts = (dst_f, dst_b, dst_u, dst_s)
    eas = (ea_f, ea_b, ea_u, ea_s)

    def run_pipe(issue, dload, nch):
        def wait_rows(rowsb, sem):
            pltpu.make_async_copy(xtab.at[pl.ds(0, C)], rowsb, sem).wait()

        def scat(rowsb, didxb):
            pltpu.sync_copy(rowsb, acc2.at[pl.ds(a2b, 10000)].at[didxb],
                            add=True)

        issue(0, sidx0, rows0, sem0)

        def body(i, _):
            issue(2 * i + 1, sidx1, rows1, sem1)
            dload(2 * i, didx0)
            wait_rows(rows0, sem0)
            scat(rows0, didx0)
            issue(2 * i + 2, sidx0, rows0, sem0)
            dload(2 * i + 1, didx1)
            wait_rows(rows1, sem1)
            scat(rows1, didx1)
            return _

        lax.fori_loop(0, nch // 2 - 1, body, 0)
        issue(nch - 1, sidx1, rows1, sem1)
        dload(nch - 2, didx0)
        wait_rows(rows0, sem0)
        scat(rows0, didx0)
        dload(nch - 1, didx1)
        wait_rows(rows1, sem1)
        scat(rows1, didx1)

    for r in range(4):
        sb = TBASE[r] * (32 * 10000) + wid * 10000
        src_r = srcs[r]
        dst_r = dsts[r]
        pltpu.sync_copy(z2, acc2.at[pl.ds(a2b, 10000), :])

        def issue(c, sidxb, rowsb, sem, sb=sb, src_r=src_r):
            pltpu.sync_copy(src_r.at[pl.ds(c * C, C)], sidxb)
            pltpu.async_copy(xtab.at[pl.ds(sb, 10000)].at[sidxb], rowsb, sem)

        def dload(c, didxb, dst_r=dst_r):
            pltpu.sync_copy(dst_r.at[pl.ds(c * C, C)], didxb)

        run_pipe(issue, dload, E // C)
        pltpu.sync_copy(acc2.at[pl.ds(a2b, 10000), :],
                        aggx.at[pl.ds(r * 10000, 10000), pl.ds(w8, 8)])

    for r in range(4):
        @pl.when(wid // 4 == r)
        def _(r=r):
            h = (wid // 2) % 2
            half = wid % 2
            eoff0 = half * E2
            ea_r = eas[r]
            dst_r = dsts[r]
            pltpu.sync_copy(z2, acc2.at[pl.ds(a2b, 10000), :])

            def issue(c, sidxb, rowsb, sem):
                pltpu.async_copy(
                    ea_r.at[pl.ds(eoff0 + c * C, C), pl.ds(h * 8, 8)],
                    rowsb, sem)

            def dload(c, didxb):
                pltpu.sync_copy(dst_r.at[pl.ds(eoff0 + c * C, C)], didxb)

            run_pipe(issue, dload, E2 // C)
            pltpu.sync_copy(
                acc2.at[pl.ds(a2b, 10000), :],
                aggeP.at[pl.ds(half * 40000 + r * 10000, 10000),
                         pl.ds(h * 8, 8)])

    for r in range(4):
        @pl.when((wid >= 16) & (wid < 24) & ((wid - 16) // 2 == r))
        def _(r=r):
            half = wid % 2
            doff0 = half * E2
            dst_r = dsts[r]
            pltpu.sync_copy(z1, acc1.at[pl.ds(a1b, 10000)])

            def fill(g, _):
                onesb[pl.ds(g * 16, 16)] = jnp.ones((16,), jnp.float32)
                return _

            lax.fori_loop(0, G, fill, 0, unroll=8)

            def chunk_body3(c, _):
                pltpu.sync_copy(dst_r.at[pl.ds(doff0 + c * C, C)], didx0)
                pltpu.sync_copy(onesb, acc1.at[pl.ds(a1b, 10000)].at[didx0],
                                add=True)
                return _

            lax.fori_loop(0, E2 // C, chunk_body3, 0)
            pltpu.sync_copy(acc1.at[pl.ds(a1b, 10000)],
                            degp.at[pl.ds((r * 2 + half) * 10000, 10000)])


_sc_aggregate = functools.partial(
    pl.kernel,
    mesh=_sc_mesh,
    compiler_params=_sc_params,
    out_type=[
        jax.ShapeDtypeStruct((4 * 10000, 256), jnp.float32),
        jax.ShapeDtypeStruct((2 * 4 * 10000, 16), jnp.float32),
        jax.ShapeDtypeStruct((8 * 10000,), jnp.float32),
    ],
    scratch_types=[
        pltpu.VMEM_SHARED((NSUB * 10000, 8), jnp.float32),
        pltpu.VMEM_SHARED((4 * 10000,), jnp.float32),
        pltpu.VMEM((C,), jnp.int32),
        pltpu.VMEM((C,), jnp.int32),
        pltpu.VMEM((C,), jnp.int32),
        pltpu.VMEM((C,), jnp.int32),
        pltpu.VMEM((C, 8), jnp.float32),
        pltpu.VMEM((C, 8), jnp.float32),
        pltpu.VMEM((C,), jnp.float32),
        pltpu.SemaphoreType.DMA,
        pltpu.SemaphoreType.DMA,
    ],
)(_sc_body)


def _p1_body(a0, e0, d0, a1, e1, d1, W0, We0, W1r, We1r, W1, b1, w2r,
             z0o, z1o, s0o, s1o, acc):
    i = pl.program_id(0)

    @pl.when(i == 0)
    def _():
        acc[0] = 0.0
        acc[1] = 0.0

    rd0 = 1.0 / jnp.maximum(d0[...], 1.0)
    rd1 = 1.0 / jnp.maximum(d1[...], 1.0)
    z0 = (jnp.dot(a0[...], W0[...], preferred_element_type=jnp.float32)
          + jnp.dot(e0[...], We0[...], preferred_element_type=jnp.float32)) * rd0
    z1 = (jnp.dot(a1[...], W1r[...], preferred_element_type=jnp.float32)
          + jnp.dot(e1[...], We1r[...], preferred_element_type=jnp.float32)) * rd1
    h0 = jnp.tanh(jnp.dot(z0, W1[...], preferred_element_type=jnp.float32) + b1[...])
    h1 = jnp.tanh(jnp.dot(z1, W1[...], preferred_element_type=jnp.float32) + b1[...])
    acc[0] += jnp.sum(h0 * w2r[...])
    acc[1] += jnp.sum(h1 * w2r[...])
    z0o[...] = z0
    z1o[...] = z1

    @pl.when(i == NBLK - 1)
    def _():
        s0o[...] = jnp.full((1, 128), acc[0], jnp.float32)
        s1o[...] = jnp.full((1, 128), acc[1], jnp.float32)


def _p2_body(z0, z1, s0, s1, o):
    t0 = s0[0, 0] * (1.0 / NU)
    t1 = s1[0, 0] * (1.0 / NU)
    m = jnp.maximum(t0, t1)
    e0 = jnp.exp(t0 - m)
    e1 = jnp.exp(t1 - m)
    b0 = e0 / (e0 + e1)
    b1 = e1 / (e0 + e1)
    o[...] = b0 * z0[...] + b1 * z1[...]


def _dense_pair(a0, e0, d0, a1, e1, d1, W0, We0, W1r, We1r, W1, b1, w2):
    row = lambda i: (i, 0)
    const = lambda i: (0, 0)
    z0, z1, s0, s1 = pl.pallas_call(
        _p1_body,
        grid=(NBLK,),
        in_specs=[
            pl.BlockSpec((NB, D), row),
            pl.BlockSpec((NB, DEA), row),
            pl.BlockSpec((NB, 1), row),
            pl.BlockSpec((NB, D), row),
            pl.BlockSpec((NB, DEA), row),
            pl.BlockSpec((NB, 1), row),
            pl.BlockSpec((D, D), const),
            pl.BlockSpec((DEA, D), const),
            pl.BlockSpec((D, D), const),
            pl.BlockSpec((DEA, D), const),
            pl.BlockSpec((D, HID), const),
            pl.BlockSpec((1, HID), const),
            pl.BlockSpec((1, HID), const),
        ],
        out_specs=[
            pl.BlockSpec((NB, D), row),
            pl.BlockSpec((NB, D), row),
            pl.BlockSpec((1, 128), const),
            pl.BlockSpec((1, 128), const),
        ],
        out_shape=[
            jax.ShapeDtypeStruct((NU, D), jnp.float32),
            jax.ShapeDtypeStruct((NU, D), jnp.float32),
            jax.ShapeDtypeStruct((1, 128), jnp.float32),
            jax.ShapeDtypeStruct((1, 128), jnp.float32),
        ],
        scratch_shapes=[pltpu.SMEM((2,), jnp.float32)],
    )(a0, e0, d0.reshape(NU, 1), a1, e1, d1.reshape(NU, 1),
      W0, We0, W1r, We1r, W1, b1.reshape(1, HID), w2.reshape(1, HID))

    out = pl.pallas_call(
        _p2_body,
        grid=(NBLK,),
        in_specs=[
            pl.BlockSpec((NB, D), row),
            pl.BlockSpec((NB, D), row),
            pl.BlockSpec((1, 128), const),
            pl.BlockSpec((1, 128), const),
        ],
        out_specs=pl.BlockSpec((NB, D), row),
        out_shape=jax.ShapeDtypeStruct((NU, D), jnp.float32),
    )(z0, z1, s0, s1)
    return out.reshape(NU, H, DH)


def kernel(x_user, x_item, ei_follows, ei_boughtby, ei_buys, ei_similar,
           ea_follows, ea_boughtby, ea_buys, ea_similar,
           W_follows, We_follows, W_boughtby, We_boughtby,
           W_buys, We_buys, W_similar, We_similar,
           W1_u, b1_u, w2_u, W1_i, b1_i, w2_i):
    xu3 = x_user.reshape(NU, 32, 8).transpose(1, 0, 2).reshape(-1, 8)
    xi3 = x_item.reshape(NI, 32, 8).transpose(1, 0, 2).reshape(-1, 8)
    xtab = jnp.concatenate([xu3, xi3], axis=0)

    z2 = jnp.zeros((10000, 8), jnp.float32)
    z1 = jnp.zeros((10000,), jnp.float32)

    aggx, aggeP, degp = _sc_aggregate(
        xtab,
        ei_follows[0], ei_follows[1], ei_boughtby[0], ei_boughtby[1],
        ei_buys[0], ei_buys[1], ei_similar[0], ei_similar[1],
        ea_follows, ea_boughtby, ea_buys, ea_similar, z2, z1)

    aggx4 = aggx.reshape(4, NU, D)
    agge4 = aggeP.reshape(2, 4, NU, DEA).sum(axis=0)
    deg4 = degp.reshape(4, 2, NU).sum(axis=1)

    out_user = _dense_pair(aggx4[0], agge4[0], deg4[0],
                           aggx4[1], agge4[1], deg4[1],
                           W_follows, We_follows, W_boughtby, We_boughtby,
                           W1_u, b1_u, w2_u)
    out_item = _dense_pair(aggx4[2], agge4[2], deg4[2],
                           aggx4[3], agge4[3], deg4[3],
                           W_buys, We_buys, W_similar, We_similar,
                           W1_i, b1_i, w2_i)
    return (out_user, out_item)
